# baseline (speedup 1.0000x reference)
import functools

import jax
import jax.numpy as jnp
from jax import lax
from jax.experimental import pallas as pl
from jax.experimental.pallas import tpu as pltpu
from jax.experimental.pallas import tpu_sc as plsc

NC = 2
NS = 16
NW = NC * NS
G = 32
NBUF = 2


@functools.partial(jax.jit, static_argnums=(2, 3))
def _gather_sc(g4, tables_gran, rows, d):
    ngroups = rows // G
    npw = ngroups // NW
    K = G * 4

    mesh = plsc.VectorSubcoreMesh(core_axis_name="c", subcore_axis_name="s")

    @functools.partial(
        pl.kernel,
        out_type=jax.ShapeDtypeStruct((rows * 4, 16), jnp.float32),
        mesh=mesh,
        compiler_params=pltpu.CompilerParams(use_tc_tiling_on_sc=False),
        scratch_types=[
            pltpu.VMEM((npw * K,), jnp.int32),
            pltpu.VMEM((NBUF, K, 16), jnp.float32),
            pltpu.SemaphoreType.DMA,
            pltpu.SemaphoreType.DMA,
        ],
    )
    def sc_kernel(g_hbm, tab_hbm, out_hbm, idx_v, rows_v, sem0, sem1):
        sems = [sem0, sem1]
        wid = lax.axis_index("s") * NC + lax.axis_index("c")
        g0 = wid * npw

        pltpu.sync_copy(g_hbm.at[pl.ds(g0 * K, npw * K)], idx_v)

        for b in range(NBUF):
            pltpu.async_copy(
                tab_hbm.at[idx_v.at[pl.ds(b * K, K)]], rows_v.at[b], sems[b])

        @pl.loop(0, npw, step=NBUF)
        def _(j0):
            for b in range(NBUF):
                j = j0 + b
                pltpu.make_async_copy(
                    tab_hbm.at[idx_v.at[pl.ds(j * K, K)]],
                    rows_v.at[b], sems[b]).wait()
                pltpu.sync_copy(
                    rows_v.at[b], out_hbm.at[pl.ds((g0 + j) * K, K)])
                @pl.when(j + NBUF < npw)
                def _():
                    pltpu.async_copy(
                        tab_hbm.at[idx_v.at[pl.ds((j + NBUF) * K, K)]],
                        rows_v.at[b], sems[b])

    return sc_kernel(g4, tables_gran)


def kernel(indices, tables):
    f, b = indices.shape
    _, v, d = tables.shape
    rows = f * b
    assert rows % (NW * G) == 0

    tables_gran = tables.reshape(f * v * 4, 16)
    offs = (jnp.arange(f, dtype=jnp.int32) * v)[None, :]
    g1 = (indices.T + offs).reshape(rows)
    g4 = (g1[:, None] * 4 + jnp.arange(4, dtype=jnp.int32)[None, :]).reshape(-1)

    out = _gather_sc(g4, tables_gran, rows, d)
    return out.reshape(b, f, d)


# speedup vs baseline: 1.0283x; 1.0283x over previous
import functools

import jax
import jax.numpy as jnp
from jax import lax
from jax.experimental import pallas as pl
from jax.experimental.pallas import tpu as pltpu
from jax.experimental.pallas import tpu_sc as plsc

NC = 2
NS = 16
NW = NC * NS
BC = 128
NBUF = 2


@functools.partial(jax.jit, static_argnums=(2, 3, 4))
def _emb_sc(indices, tgran, f_n, v_n, d_n):
    b_n = indices.shape[1]
    kn = d_n // 16
    assert b_n == BC * NW and kn == 4

    mesh = plsc.VectorSubcoreMesh(core_axis_name="c", subcore_axis_name="s")

    @functools.partial(
        pl.kernel,
        out_type=jax.ShapeDtypeStruct((b_n, f_n, d_n), jnp.float32),
        mesh=mesh,
        compiler_params=pltpu.CompilerParams(use_tc_tiling_on_sc=False),
        scratch_types=[
            pltpu.VMEM((f_n, BC), jnp.int32),
            pltpu.VMEM((f_n, kn, BC), jnp.int32),
            pltpu.VMEM((NBUF, kn, BC, 16), jnp.float32),
            pltpu.SemaphoreType.DMA,
            pltpu.SemaphoreType.DMA,
        ],
    )
    def sc_kernel(idx_hbm, tg_hbm, out_hbm, idx_v, gid_v, rows_v, sem0, sem1):
        sems = [sem0, sem1]
        wid = lax.axis_index("s") * NC + lax.axis_index("c")
        b0 = wid * BC

        pltpu.sync_copy(idx_hbm.at[:, pl.ds(b0, BC)], idx_v)

        @pl.loop(0, f_n)
        def _(f):
            fbase = f * v_n

            @pl.loop(0, BC // 16, unroll=4)
            def _(j):
                g4 = (idx_v[f, pl.ds(j * 16, 16)] + fbase) << 2
                for k in range(kn):
                    gid_v[f, k, pl.ds(j * 16, 16)] = g4 + k

        def fire(f, b):
            for k in range(kn):
                pltpu.async_copy(tg_hbm.at[gid_v.at[f, k]],
                                 rows_v.at[b, k], sems[b])

        def drain(f, b):
            for k in range(kn):
                pltpu.make_async_copy(tg_hbm.at[gid_v.at[f, k]],
                                      rows_v.at[b, k], sems[b]).wait()

        for b in range(NBUF):
            fire(b, b)

        @pl.loop(0, f_n, step=NBUF)
        def _(f0):
            for b in range(NBUF):
                f = f0 + b
                drain(f, b)
                for k in range(kn):
                    pltpu.sync_copy(
                        rows_v.at[b, k],
                        out_hbm.at[pl.ds(b0, BC), f, pl.ds(k * 16, 16)])

                @pl.when(f + NBUF < f_n)
                def _():
                    fire(f + NBUF, b)

    return sc_kernel(indices, tgran)


def kernel(indices, tables):
    f, b = indices.shape
    _, v, d = tables.shape
    assert b == BC * NW and d == 64

    tgran = tables.reshape(f * v * d // 16, 16)
    return _emb_sc(indices, tgran, f, v, d)
